# Initial kernel scaffold; baseline (speedup 1.0000x reference)
import functools

import jax
import jax.numpy as jnp
from jax import lax
from jax.experimental import pallas as pl
from jax.experimental.pallas import tpu as pltpu
from jax.experimental.pallas import tpu_sc as plsc

_NC = 2
_NS = 16
_NW = _NC * _NS

_SUB = 128
_NSUB = 10
_CHUNK = _SUB * _NSUB


@functools.lru_cache(maxsize=None)
def _make_gather(B, D):
    b_per_w = B // _NW
    assert B % _NW == 0 and b_per_w % _CHUNK == 0
    n_steps = b_per_w // _CHUNK
    mesh = plsc.VectorSubcoreMesh(core_axis_name="c", subcore_axis_name="s")

    @functools.partial(
        pl.kernel,
        out_type=jax.ShapeDtypeStruct((B, D), jnp.float32),
        mesh=mesh,
        scratch_types=[
            pltpu.VMEM((_CHUNK,), jnp.int32),
            pltpu.VMEM((_CHUNK, D), jnp.float32),
            pltpu.SemaphoreType.DMA,
        ],
    )
    def gather_kernel(idx_hbm, table_hbm, out_hbm, idx_v, rows_v, sem):
        wid = lax.axis_index("s") * _NC + lax.axis_index("c")
        base = wid * b_per_w

        def step(g, carry):
            off = base + g * _CHUNK
            pltpu.sync_copy(idx_hbm.at[pl.ds(off, _CHUNK)], idx_v)
            copies = [
                pltpu.async_copy(
                    table_hbm.at[idx_v.at[pl.ds(j * _SUB, _SUB)]],
                    rows_v.at[pl.ds(j * _SUB, _SUB)],
                    sem,
                )
                for j in range(_NSUB)
            ]
            for c in copies:
                c.wait()
            pltpu.sync_copy(rows_v, out_hbm.at[pl.ds(off, _CHUNK)])
            return carry

        lax.fori_loop(0, n_steps, step, 0)

    return gather_kernel


def kernel(x, embedding):
    batch, hist = x.shape
    depth = embedding.shape[1]
    flat = x.reshape(batch * hist)
    out = _make_gather(batch * hist, depth)(flat, embedding)
    return out.reshape(batch, hist, depth)


# speedup vs baseline: 1.4707x; 1.4707x over previous
import functools

import jax
import jax.numpy as jnp
from jax import lax
from jax.experimental import pallas as pl
from jax.experimental.pallas import tpu as pltpu
from jax.experimental.pallas import tpu_sc as plsc

_NC = 2
_NS = 16
_NW = _NC * _NS

_SUB = 128
_NSUB = 10
_CHUNK = _SUB * _NSUB


@functools.lru_cache(maxsize=None)
def _make_gather(B, D):
    b_per_w = B // _NW
    assert B % _NW == 0 and b_per_w % _CHUNK == 0
    n_steps = b_per_w // _CHUNK
    mesh = plsc.VectorSubcoreMesh(core_axis_name="c", subcore_axis_name="s")

    @functools.partial(
        pl.kernel,
        out_type=jax.ShapeDtypeStruct((B, D), jnp.float32),
        mesh=mesh,
        scratch_types=[
            pltpu.VMEM((_CHUNK,), jnp.int32),
            pltpu.VMEM((_CHUNK, D), jnp.float32),
            pltpu.SemaphoreType.DMA,
        ],
        compiler_params=pltpu.CompilerParams(use_tc_tiling_on_sc=False),
    )
    def gather_kernel(idx_hbm, table_hbm, out_hbm, idx_v, rows_v, sem):
        wid = lax.axis_index("s") * _NC + lax.axis_index("c")
        base = wid * b_per_w

        def step(g, carry):
            off = base + g * _CHUNK
            pltpu.sync_copy(idx_hbm.at[pl.ds(off, _CHUNK)], idx_v)
            copies = [
                pltpu.async_copy(
                    table_hbm.at[idx_v.at[pl.ds(j * _SUB, _SUB)]],
                    rows_v.at[pl.ds(j * _SUB, _SUB)],
                    sem,
                )
                for j in range(_NSUB)
            ]
            for c in copies:
                c.wait()
            pltpu.sync_copy(rows_v, out_hbm.at[pl.ds(off, _CHUNK)])
            return carry

        lax.fori_loop(0, n_steps, step, 0)

    return gather_kernel


def kernel(x, embedding):
    batch, hist = x.shape
    depth = embedding.shape[1]
    flat = x.reshape(batch * hist)
    out = _make_gather(batch * hist, depth)(flat, embedding)
    return out.reshape(batch, hist, depth)


# speedup vs baseline: 1.4968x; 1.0177x over previous
import functools

import jax
import jax.numpy as jnp
from jax import lax
from jax.experimental import pallas as pl
from jax.experimental.pallas import tpu as pltpu
from jax.experimental.pallas import tpu_sc as plsc

_NC = 2
_NS = 16
_NW = _NC * _NS

_SUB = 128
_NSUB = 10
_CHUNK = _SUB * _NSUB


@functools.lru_cache(maxsize=None)
def _make_gather(B, D):
    b_per_w = B // _NW
    assert B % _NW == 0 and b_per_w % (2 * _CHUNK) == 0
    n_pairs = b_per_w // (2 * _CHUNK)
    mesh = plsc.VectorSubcoreMesh(core_axis_name="c", subcore_axis_name="s")

    @functools.partial(
        pl.kernel,
        out_type=jax.ShapeDtypeStruct((B, D), jnp.float32),
        mesh=mesh,
        scratch_types=[
            pltpu.VMEM((_CHUNK,), jnp.int32),
            pltpu.VMEM((_CHUNK,), jnp.int32),
            pltpu.VMEM((_CHUNK, D), jnp.float32),
            pltpu.VMEM((_CHUNK, D), jnp.float32),
            pltpu.SemaphoreType.DMA,
            pltpu.SemaphoreType.DMA,
            pltpu.SemaphoreType.DMA,
            pltpu.SemaphoreType.DMA,
        ],
        compiler_params=pltpu.CompilerParams(use_tc_tiling_on_sc=False),
    )
    def gather_kernel(idx_hbm, table_hbm, out_hbm, idx0, idx1, rows0, rows1,
                      gsem0, gsem1, osem0, osem1):
        bufs = ((idx0, rows0, gsem0, osem0), (idx1, rows1, gsem1, osem1))
        wid = lax.axis_index("s") * _NC + lax.axis_index("c")
        base = wid * b_per_w

        def pair(i, carry):
            pending = []
            for b in range(2):
                idx_b, rows_b, gsem_b, osem_b = bufs[b]
                off = base + (2 * i + b) * _CHUNK

                @pl.when(i > 0)
                def _():
                    pltpu.make_async_copy(
                        rows_b,
                        out_hbm.at[pl.ds(off - 2 * _CHUNK, _CHUNK)],
                        osem_b,
                    ).wait()

                pltpu.sync_copy(idx_hbm.at[pl.ds(off, _CHUNK)], idx_b)
                pending.append([
                    pltpu.async_copy(
                        table_hbm.at[idx_b.at[pl.ds(j * _SUB, _SUB)]],
                        rows_b.at[pl.ds(j * _SUB, _SUB)],
                        gsem_b,
                    )
                    for j in range(_NSUB)
                ])
            for b in range(2):
                idx_b, rows_b, gsem_b, osem_b = bufs[b]
                off = base + (2 * i + b) * _CHUNK
                for c in pending[b]:
                    c.wait()
                pltpu.async_copy(rows_b, out_hbm.at[pl.ds(off, _CHUNK)], osem_b)
            return carry

        lax.fori_loop(0, n_pairs, pair, 0)

        for b in range(2):
            _, rows_b, _, osem_b = bufs[b]
            off = base + (2 * (n_pairs - 1) + b) * _CHUNK
            pltpu.make_async_copy(
                rows_b, out_hbm.at[pl.ds(off, _CHUNK)], osem_b
            ).wait()

    return gather_kernel


def kernel(x, embedding):
    batch, hist = x.shape
    depth = embedding.shape[1]
    flat = x.reshape(batch * hist)
    out = _make_gather(batch * hist, depth)(flat, embedding)
    return out.reshape(batch, hist, depth)
